# Initial kernel scaffold; baseline (speedup 1.0000x reference)
import jax
import jax.numpy as jnp
from jax.experimental import pallas as pl


def kernel(x, edge_index, W_in, b_in, Wl0, bl0, Wr0, Wl1, bl1, Wr1, Wl2, bl2, Wr2):
    raise NotImplementedError("write your pallas kernel here")


# speedup vs baseline: 6.0359x; 6.0359x over previous
import jax
import jax.numpy as jnp
from jax import lax
from jax.experimental import pallas as pl
from jax.experimental.pallas import tpu as pltpu
from jax.experimental.pallas import tpu_sc as plsc

N = 10000
E = 320000
D = 128
NC = 2
NS = 16
NW = NC * NS
EPW = E // NW
CH = 80
NCHUNK = EPW // CH
ROWS_PT = 624
REM_ROWS = N - NS * ROWS_PT


def _seg_body(h_hbm, src_hbm, dst_hbm, zeros_hbm, agg_out,
              src_big, dst_big, srcs, dsts, rows, agg_sh, gsem):
    c = lax.axis_index("c")
    s = lax.axis_index("s")
    wid = s * NC + c
    base = wid * EPW
    start = s * ROWS_PT

    pltpu.sync_copy(zeros_hbm.at[pl.ds(start, ROWS_PT)],
                    agg_sh.at[pl.ds(start, ROWS_PT)])

    @pl.when(s == NS - 1)
    def _zero_rem():
        pltpu.sync_copy(zeros_hbm.at[pl.ds(NS * ROWS_PT, REM_ROWS)],
                        agg_sh.at[pl.ds(NS * ROWS_PT, REM_ROWS)])

    pltpu.sync_copy(src_hbm.at[pl.ds(base, EPW)], src_big)
    pltpu.sync_copy(dst_hbm.at[pl.ds(base, EPW)], dst_big)
    plsc.subcore_barrier()

    def chunk(j, carry):
        off = j * CH
        for k in range(CH // 16):
            srcs[pl.ds(k * 16, 16)] = src_big[pl.ds(off + k * 16, 16)]
            dsts[pl.ds(k * 16, 16)] = dst_big[pl.ds(off + k * 16, 16)]
        pltpu.async_copy(h_hbm.at[srcs], rows, gsem).wait()
        pltpu.sync_copy(rows, agg_sh.at[dsts], add=True)
        return carry

    lax.fori_loop(0, NCHUNK, chunk, 0)

    plsc.subcore_barrier()
    pltpu.sync_copy(agg_sh.at[pl.ds(start, ROWS_PT)],
                    agg_out.at[c, pl.ds(start, ROWS_PT)])

    @pl.when(s == NS - 1)
    def _read_rem():
        pltpu.sync_copy(agg_sh.at[pl.ds(NS * ROWS_PT, REM_ROWS)],
                        agg_out.at[c, pl.ds(NS * ROWS_PT, REM_ROWS)])


def _deg_body(dst_hbm, zeros_hbm, ones_hbm, deg_out,
              dst_big, dsts, ones_v, deg_sh):
    c = lax.axis_index("c")
    s = lax.axis_index("s")
    wid = s * NC + c
    base = wid * EPW
    start = s * ROWS_PT

    pltpu.sync_copy(zeros_hbm.at[pl.ds(start, ROWS_PT)],
                    deg_sh.at[pl.ds(start, ROWS_PT)])

    @pl.when(s == NS - 1)
    def _zero_rem():
        pltpu.sync_copy(zeros_hbm.at[pl.ds(NS * ROWS_PT, REM_ROWS)],
                        deg_sh.at[pl.ds(NS * ROWS_PT, REM_ROWS)])

    pltpu.sync_copy(ones_hbm, ones_v)
    pltpu.sync_copy(dst_hbm.at[pl.ds(base, EPW)], dst_big)
    plsc.subcore_barrier()

    def chunk(j, carry):
        off = j * CH
        for k in range(CH // 16):
            dsts[pl.ds(k * 16, 16)] = dst_big[pl.ds(off + k * 16, 16)]
        pltpu.sync_copy(ones_v, deg_sh.at[dsts], add=True)
        return carry

    lax.fori_loop(0, NCHUNK, chunk, 0)

    plsc.subcore_barrier()
    pltpu.sync_copy(deg_sh.at[pl.ds(start, ROWS_PT)],
                    deg_out.at[c, pl.ds(start, ROWS_PT)])

    @pl.when(s == NS - 1)
    def _read_rem():
        pltpu.sync_copy(deg_sh.at[pl.ds(NS * ROWS_PT, REM_ROWS)],
                        deg_out.at[c, pl.ds(NS * ROWS_PT, REM_ROWS)])


_sc_kernel_cache = {}


def _get_seg_kernel():
    if "seg" not in _sc_kernel_cache:
        mesh = plsc.VectorSubcoreMesh(
            core_axis_name="c", subcore_axis_name="s",
            num_cores=NC, num_subcores=NS)
        _sc_kernel_cache["seg"] = pl.kernel(
            _seg_body,
            out_type=jax.ShapeDtypeStruct((NC, N, D), jnp.float32),
            mesh=mesh,
            scratch_types=(
                pltpu.VMEM((EPW,), jnp.int32),
                pltpu.VMEM((EPW,), jnp.int32),
                pltpu.VMEM((CH,), jnp.int32),
                pltpu.VMEM((CH,), jnp.int32),
                pltpu.VMEM((CH, D), jnp.float32),
                pltpu.VMEM_SHARED((N, D), jnp.float32),
                pltpu.SemaphoreType.DMA,
            ),
            name="sage_segment_sum_sc",
        )
    return _sc_kernel_cache["seg"]


def _get_deg_kernel():
    if "deg" not in _sc_kernel_cache:
        mesh = plsc.VectorSubcoreMesh(
            core_axis_name="c", subcore_axis_name="s",
            num_cores=NC, num_subcores=NS)
        _sc_kernel_cache["deg"] = pl.kernel(
            _deg_body,
            out_type=jax.ShapeDtypeStruct((NC, N, D), jnp.float32),
            mesh=mesh,
            scratch_types=(
                pltpu.VMEM((EPW,), jnp.int32),
                pltpu.VMEM((CH,), jnp.int32),
                pltpu.VMEM((CH, D), jnp.float32),
                pltpu.VMEM_SHARED((N, D), jnp.float32),
            ),
            name="sage_degree_sc",
        )
    return _sc_kernel_cache["deg"]


BR = 2000


def _pre_body(x_ref, w_ref, b_ref, inp_ref, hr_ref):
    h = jnp.dot(x_ref[...], w_ref[...], preferred_element_type=jnp.float32)
    h = h + b_ref[...]
    inp_ref[...] = h
    hr_ref[...] = jnp.maximum(h, 0.0)


_pre = pl.pallas_call(
    _pre_body,
    grid=(N // BR,),
    in_specs=[pl.BlockSpec((BR, D), lambda i: (i, 0)),
              pl.BlockSpec((D, D), lambda i: (0, 0)),
              pl.BlockSpec((1, D), lambda i: (0, 0))],
    out_specs=[pl.BlockSpec((BR, D), lambda i: (i, 0)),
               pl.BlockSpec((BR, D), lambda i: (i, 0))],
    out_shape=[jax.ShapeDtypeStruct((N, D), jnp.float32),
               jax.ShapeDtypeStruct((N, D), jnp.float32)],
    name="sage_in_proj_tc",
)


def _combine_mid_body(aggA, aggB, degA, degB, h, inp, wl, bl, wr, out):
    deg = degA[...][:, 0:1] + degB[...][:, 0:1]
    inv = 1.0 / jnp.maximum(deg, 1.0)
    agg = (aggA[...] + aggB[...]) * inv
    o = (jnp.dot(agg, wl[...], preferred_element_type=jnp.float32)
         + jnp.dot(h[...], wr[...], preferred_element_type=jnp.float32)
         + bl[...])
    out[...] = jnp.maximum(o, 0.0) + 0.2 * inp[...]


def _combine_last_body(aggA, aggB, degA, degB, h, wl, bl, wr, out):
    deg = degA[...][:, 0:1] + degB[...][:, 0:1]
    inv = 1.0 / jnp.maximum(deg, 1.0)
    agg = (aggA[...] + aggB[...]) * inv
    o = (jnp.dot(agg, wl[...], preferred_element_type=jnp.float32)
         + jnp.dot(h[...], wr[...], preferred_element_type=jnp.float32)
         + bl[...])
    m = jnp.max(o, axis=-1, keepdims=True)
    sh = o - m
    lse = jnp.log(jnp.sum(jnp.exp(sh), axis=-1, keepdims=True))
    out[...] = sh - lse


_row_spec = pl.BlockSpec((BR, D), lambda i: (i, 0))
_w_spec = pl.BlockSpec((D, D), lambda i: (0, 0))
_b_spec = pl.BlockSpec((1, D), lambda i: (0, 0))

_combine_mid = pl.pallas_call(
    _combine_mid_body,
    grid=(N // BR,),
    in_specs=[_row_spec, _row_spec, _row_spec, _row_spec, _row_spec,
              _row_spec, _w_spec, _b_spec, _w_spec],
    out_specs=_row_spec,
    out_shape=jax.ShapeDtypeStruct((N, D), jnp.float32),
    name="sage_combine_mid_tc",
)

_combine_last = pl.pallas_call(
    _combine_last_body,
    grid=(N // BR,),
    in_specs=[_row_spec, _row_spec, _row_spec, _row_spec, _row_spec,
              _w_spec, _b_spec, _w_spec],
    out_specs=_row_spec,
    out_shape=jax.ShapeDtypeStruct((N, D), jnp.float32),
    name="sage_combine_last_tc",
)


def kernel(x, edge_index, W_in, b_in, Wl0, bl0, Wr0, Wl1, bl1, Wr1, Wl2, bl2,
           Wr2):
    src = edge_index[0].astype(jnp.int32)
    dst = edge_index[1].astype(jnp.int32)
    zeros_big = jnp.zeros((N, D), jnp.float32)
    ones_ch = jnp.ones((CH, D), jnp.float32)

    inp, h = _pre(x, W_in, b_in.reshape(1, D))
    _seg = _get_seg_kernel()
    _deg = _get_deg_kernel()

    degp = _deg(dst, zeros_big, ones_ch)

    aggp = _seg(h, src, dst, zeros_big)
    h = _combine_mid(aggp[0], aggp[1], degp[0], degp[1], h, inp,
                     Wl0, bl0.reshape(1, D), Wr0)

    aggp = _seg(h, src, dst, zeros_big)
    h = _combine_mid(aggp[0], aggp[1], degp[0], degp[1], h, inp,
                     Wl1, bl1.reshape(1, D), Wr1)

    aggp = _seg(h, src, dst, zeros_big)
    out = _combine_last(aggp[0], aggp[1], degp[0], degp[1], h,
                        Wl2, bl2.reshape(1, D), Wr2)
    return out


# speedup vs baseline: 8.9516x; 1.4831x over previous
import jax
import jax.numpy as jnp
from jax import lax
from jax.experimental import pallas as pl
from jax.experimental.pallas import tpu as pltpu
from jax.experimental.pallas import tpu_sc as plsc

N = 10000
E = 320000
D = 128
NC = 2
NS = 16
NW = NC * NS
EPW = E // NW
CH = 80
NCHUNK = EPW // CH
NBUF = 2
NGRP = (NCHUNK - 1) // NBUF
ROWS_PT = 624
REM_ROWS = N - NS * ROWS_PT


def _seg_body(h_hbm, src_hbm, dst_hbm, zeros_hbm, agg_out,
              src_big, dst_big, s0, s1, d0, d1, r0, r1, agg_sh,
              g0, g1, t0, t1):
    srcb = [s0, s1]
    dstb = [d0, d1]
    rows = [r0, r1]
    gs = [g0, g1]
    ts = [t0, t1]
    c = lax.axis_index("c")
    s = lax.axis_index("s")
    wid = s * NC + c
    base = wid * EPW
    start = s * ROWS_PT

    pltpu.sync_copy(zeros_hbm.at[pl.ds(start, ROWS_PT)],
                    agg_sh.at[pl.ds(start, ROWS_PT)])

    @pl.when(s == NS - 1)
    def _zero_rem():
        pltpu.sync_copy(zeros_hbm.at[pl.ds(NS * ROWS_PT, REM_ROWS)],
                        agg_sh.at[pl.ds(NS * ROWS_PT, REM_ROWS)])

    pltpu.sync_copy(src_hbm.at[pl.ds(base, EPW)], src_big)
    pltpu.sync_copy(dst_hbm.at[pl.ds(base, EPW)], dst_big)
    plsc.subcore_barrier()

    def fill(b, j):
        off = j * CH
        for k in range(CH // 16):
            srcb[b][pl.ds(k * 16, 16)] = src_big[pl.ds(off + k * 16, 16)]
            dstb[b][pl.ds(k * 16, 16)] = dst_big[pl.ds(off + k * 16, 16)]

    for b in range(NBUF):
        fill(b, b)
        pltpu.async_copy(h_hbm.at[srcb[b]], rows[b], gs[b])

    def group(g, carry):
        for b in range(NBUF):
            pltpu.make_async_copy(h_hbm.at[srcb[b]], rows[b], gs[b]).wait()
            pltpu.async_copy(rows[b], agg_sh.at[dstb[b]], ts[b], add=True)
        for b in range(NBUF):
            pltpu.make_async_copy(rows[b], agg_sh.at[dstb[b]], ts[b]).wait()
            fill(b, (g + 1) * NBUF + b)
            pltpu.async_copy(h_hbm.at[srcb[b]], rows[b], gs[b])
        return carry

    lax.fori_loop(0, NGRP - 1, group, 0)

    for b in range(NBUF):
        pltpu.make_async_copy(h_hbm.at[srcb[b]], rows[b], gs[b]).wait()
        pltpu.async_copy(rows[b], agg_sh.at[dstb[b]], ts[b], add=True)
    for b in range(NBUF):
        pltpu.make_async_copy(rows[b], agg_sh.at[dstb[b]], ts[b]).wait()

    fill(0, NCHUNK - 1)
    pltpu.async_copy(h_hbm.at[srcb[0]], rows[0], gs[0]).wait()
    pltpu.sync_copy(rows[0], agg_sh.at[dstb[0]], add=True)

    plsc.subcore_barrier()
    pltpu.sync_copy(agg_sh.at[pl.ds(start, ROWS_PT)],
                    agg_out.at[c, pl.ds(start, ROWS_PT)])

    @pl.when(s == NS - 1)
    def _read_rem():
        pltpu.sync_copy(agg_sh.at[pl.ds(NS * ROWS_PT, REM_ROWS)],
                        agg_out.at[c, pl.ds(NS * ROWS_PT, REM_ROWS)])


def _deg_body(dst_hbm, zeros_hbm, ones_hbm, deg_out,
              dst_big, dsts, ones_v, deg_sh):
    c = lax.axis_index("c")
    s = lax.axis_index("s")
    wid = s * NC + c
    base = wid * EPW
    start = s * ROWS_PT

    pltpu.sync_copy(zeros_hbm.at[pl.ds(start, ROWS_PT)],
                    deg_sh.at[pl.ds(start, ROWS_PT)])

    @pl.when(s == NS - 1)
    def _zero_rem():
        pltpu.sync_copy(zeros_hbm.at[pl.ds(NS * ROWS_PT, REM_ROWS)],
                        deg_sh.at[pl.ds(NS * ROWS_PT, REM_ROWS)])

    pltpu.sync_copy(ones_hbm, ones_v)
    pltpu.sync_copy(dst_hbm.at[pl.ds(base, EPW)], dst_big)
    plsc.subcore_barrier()

    def chunk(j, carry):
        off = j * CH
        for k in range(CH // 16):
            dsts[pl.ds(k * 16, 16)] = dst_big[pl.ds(off + k * 16, 16)]
        pltpu.sync_copy(ones_v, deg_sh.at[dsts], add=True)
        return carry

    lax.fori_loop(0, NCHUNK, chunk, 0)

    plsc.subcore_barrier()
    pltpu.sync_copy(deg_sh.at[pl.ds(start, ROWS_PT)],
                    deg_out.at[c, pl.ds(start, ROWS_PT)])

    @pl.when(s == NS - 1)
    def _read_rem():
        pltpu.sync_copy(deg_sh.at[pl.ds(NS * ROWS_PT, REM_ROWS)],
                        deg_out.at[c, pl.ds(NS * ROWS_PT, REM_ROWS)])


_sc_kernel_cache = {}


def _get_seg_kernel():
    if "seg" not in _sc_kernel_cache:
        mesh = plsc.VectorSubcoreMesh(
            core_axis_name="c", subcore_axis_name="s",
            num_cores=NC, num_subcores=NS)
        _sc_kernel_cache["seg"] = pl.kernel(
            _seg_body,
            out_type=jax.ShapeDtypeStruct((NC, N, D), jnp.float32),
            mesh=mesh,
            scratch_types=(
                (pltpu.VMEM((EPW,), jnp.int32),) * 2
                + (pltpu.VMEM((CH,), jnp.int32),) * (2 * NBUF)
                + (pltpu.VMEM((CH, D), jnp.float32),) * NBUF
                + (pltpu.VMEM_SHARED((N, D), jnp.float32),)
                + (pltpu.SemaphoreType.DMA,) * (2 * NBUF)
            ),
            name="sage_segment_sum_sc",
        )
    return _sc_kernel_cache["seg"]


def _get_deg_kernel():
    if "deg" not in _sc_kernel_cache:
        mesh = plsc.VectorSubcoreMesh(
            core_axis_name="c", subcore_axis_name="s",
            num_cores=NC, num_subcores=NS)
        _sc_kernel_cache["deg"] = pl.kernel(
            _deg_body,
            out_type=jax.ShapeDtypeStruct((NC, N, D), jnp.float32),
            mesh=mesh,
            scratch_types=(
                pltpu.VMEM((EPW,), jnp.int32),
                pltpu.VMEM((CH,), jnp.int32),
                pltpu.VMEM((CH, D), jnp.float32),
                pltpu.VMEM_SHARED((N, D), jnp.float32),
            ),
            name="sage_degree_sc",
        )
    return _sc_kernel_cache["deg"]


BR = 2000


def _pre_body(x_ref, w_ref, b_ref, inp_ref, hr_ref):
    h = jnp.dot(x_ref[...], w_ref[...], preferred_element_type=jnp.float32)
    h = h + b_ref[...]
    inp_ref[...] = h
    hr_ref[...] = jnp.maximum(h, 0.0)


_pre = pl.pallas_call(
    _pre_body,
    grid=(N // BR,),
    in_specs=[pl.BlockSpec((BR, D), lambda i: (i, 0)),
              pl.BlockSpec((D, D), lambda i: (0, 0)),
              pl.BlockSpec((1, D), lambda i: (0, 0))],
    out_specs=[pl.BlockSpec((BR, D), lambda i: (i, 0)),
               pl.BlockSpec((BR, D), lambda i: (i, 0))],
    out_shape=[jax.ShapeDtypeStruct((N, D), jnp.float32),
               jax.ShapeDtypeStruct((N, D), jnp.float32)],
    name="sage_in_proj_tc",
)


def _combine_mid_body(aggA, aggB, degA, degB, h, inp, wl, bl, wr, out):
    deg = degA[...][:, 0:1] + degB[...][:, 0:1]
    inv = 1.0 / jnp.maximum(deg, 1.0)
    agg = (aggA[...] + aggB[...]) * inv
    o = (jnp.dot(agg, wl[...], preferred_element_type=jnp.float32)
         + jnp.dot(h[...], wr[...], preferred_element_type=jnp.float32)
         + bl[...])
    out[...] = jnp.maximum(o, 0.0) + 0.2 * inp[...]


def _combine_last_body(aggA, aggB, degA, degB, h, wl, bl, wr, out):
    deg = degA[...][:, 0:1] + degB[...][:, 0:1]
    inv = 1.0 / jnp.maximum(deg, 1.0)
    agg = (aggA[...] + aggB[...]) * inv
    o = (jnp.dot(agg, wl[...], preferred_element_type=jnp.float32)
         + jnp.dot(h[...], wr[...], preferred_element_type=jnp.float32)
         + bl[...])
    m = jnp.max(o, axis=-1, keepdims=True)
    sh = o - m
    lse = jnp.log(jnp.sum(jnp.exp(sh), axis=-1, keepdims=True))
    out[...] = sh - lse


_row_spec = pl.BlockSpec((BR, D), lambda i: (i, 0))
_w_spec = pl.BlockSpec((D, D), lambda i: (0, 0))
_b_spec = pl.BlockSpec((1, D), lambda i: (0, 0))

_combine_mid = pl.pallas_call(
    _combine_mid_body,
    grid=(N // BR,),
    in_specs=[_row_spec, _row_spec, _row_spec, _row_spec, _row_spec,
              _row_spec, _w_spec, _b_spec, _w_spec],
    out_specs=_row_spec,
    out_shape=jax.ShapeDtypeStruct((N, D), jnp.float32),
    name="sage_combine_mid_tc",
)

_combine_last = pl.pallas_call(
    _combine_last_body,
    grid=(N // BR,),
    in_specs=[_row_spec, _row_spec, _row_spec, _row_spec, _row_spec,
              _w_spec, _b_spec, _w_spec],
    out_specs=_row_spec,
    out_shape=jax.ShapeDtypeStruct((N, D), jnp.float32),
    name="sage_combine_last_tc",
)


def kernel(x, edge_index, W_in, b_in, Wl0, bl0, Wr0, Wl1, bl1, Wr1, Wl2, bl2,
           Wr2):
    src = edge_index[0].astype(jnp.int32)
    dst = edge_index[1].astype(jnp.int32)
    zeros_big = jnp.zeros((N, D), jnp.float32)
    ones_ch = jnp.ones((CH, D), jnp.float32)

    inp, h = _pre(x, W_in, b_in.reshape(1, D))
    _seg = _get_seg_kernel()
    _deg = _get_deg_kernel()

    degp = _deg(dst, zeros_big, ones_ch)

    aggp = _seg(h, src, dst, zeros_big)
    h = _combine_mid(aggp[0], aggp[1], degp[0], degp[1], h, inp,
                     Wl0, bl0.reshape(1, D), Wr0)

    aggp = _seg(h, src, dst, zeros_big)
    h = _combine_mid(aggp[0], aggp[1], degp[0], degp[1], h, inp,
                     Wl1, bl1.reshape(1, D), Wr1)

    aggp = _seg(h, src, dst, zeros_big)
    out = _combine_last(aggp[0], aggp[1], degp[0], degp[1], h,
                        Wl2, bl2.reshape(1, D), Wr2)
    return out
